# baseline (speedup 1.0000x reference)
import functools

import jax
import jax.numpy as jnp
from jax import lax
from jax.experimental import pallas as pl
from jax.experimental.pallas import tpu as pltpu
from jax.experimental.pallas import tpu_sc as plsc

_BATCH = 32
_L = 3
_TPS = _L + 1
_ROWS = _BATCH * _TPS
_HID = 1024
_VOCAB = 100000

_TILE_A = 8192
_TILE_B = 4096

_HIGH = jax.lax.Precision.HIGHEST
_BIG_I32 = 2**30


_NW = 32
_STRIPE = 3136
_CHUNK = 392
_NCH = _STRIPE // _CHUNK
_NK = 8
_VPB = _CHUNK // _NK


def _sc_argmax_body(lg_ref, pmax_ref, pidx_ref, buf0, buf1, rm_ref, ri_ref,
                    sem0, sem1):
    wid = lax.axis_index("s") * 2 + lax.axis_index("c")
    base = wid * _STRIPE
    bufs = (buf0, buf1)
    sems = (sem0, sem1)

    def chunk_start(c):
        return jnp.minimum(base + c * _CHUNK, _VOCAB - _CHUNK)

    def issue(c):
        return pltpu.async_copy(
            lg_ref.at[pl.ds(chunk_start(c), _CHUNK)],
            bufs[c % 2], sems[c % 2])

    cp = issue(0)
    for c in range(_NCH):
        cp_next = issue(c + 1) if c + 1 < _NCH else None
        cp.wait()
        buf = bufs[c % 2]
        st_c = chunk_start(c)
        for cg in range(8):
            def body(v8, carry):
                ms, ixs = carry
                ms2, ixs2 = [], []
                for k in range(_NK):
                    row = v8 * _NK + k
                    x = buf[row, pl.ds(cg * 16, 16)]
                    vs = jnp.zeros((16,), jnp.int32) + (st_c + row)
                    upd = x > ms[k]
                    ms2.append(jnp.where(upd, x, ms[k]))
                    ixs2.append(jnp.where(upd, vs, ixs[k]))
                return (tuple(ms2), tuple(ixs2))

            init = (tuple(jnp.full((16,), -jnp.inf, jnp.float32)
                          for _ in range(_NK)),
                    tuple(jnp.zeros((16,), jnp.int32) for _ in range(_NK)))
            ms, ixs = jax.lax.fori_loop(0, _VPB, body, init)
            for k in range(_NK):
                off = ((c * 8 + cg) * _NK + k) * 16
                rm_ref[pl.ds(off, 16)] = ms[k]
                ri_ref[pl.ds(off, 16)] = ixs[k]
        cp = cp_next

    for cg in range(8):
        m = jnp.full((16,), -jnp.inf, jnp.float32)
        ix = jnp.zeros((16,), jnp.int32)
        for c in range(_NCH):
            for k in range(_NK):
                off = ((c * 8 + cg) * _NK + k) * 16
                m2 = rm_ref[pl.ds(off, 16)]
                i2 = ri_ref[pl.ds(off, 16)]
                gt = m2 > m
                eq = m2 == m
                ix = jnp.where(gt, i2, jnp.where(eq, jnp.minimum(ix, i2), ix))
                m = jnp.where(gt, m2, m)
        rm_ref[pl.ds(cg * 16, 16)] = m
        ri_ref[pl.ds(cg * 16, 16)] = ix

    pltpu.sync_copy(rm_ref.at[pl.ds(0, _ROWS)], pmax_ref.at[wid])
    pltpu.sync_copy(ri_ref.at[pl.ds(0, _ROWS)], pidx_ref.at[wid])


def _sc_argmax(logits_t):
    mesh = plsc.VectorSubcoreMesh(core_axis_name="c", subcore_axis_name="s")
    nacc = _NCH * 8 * _NK * 16
    f = functools.partial(
        pl.kernel,
        mesh=mesh,
        out_type=[
            jax.ShapeDtypeStruct((_NW, _ROWS), jnp.float32),
            jax.ShapeDtypeStruct((_NW, _ROWS), jnp.int32),
        ],
        scratch_types=[
            pltpu.VMEM((_CHUNK, _ROWS), jnp.float32),
            pltpu.VMEM((_CHUNK, _ROWS), jnp.float32),
            pltpu.VMEM((nacc,), jnp.float32),
            pltpu.VMEM((nacc,), jnp.int32),
            pltpu.SemaphoreType.DMA,
            pltpu.SemaphoreType.DMA,
        ],
    )(_sc_argmax_body)
    return f(logits_t)


def _phase_a_body(nva, dp_ref, logits_ref, gid_ref, nacc_ref, last_ref,
                  vmax_ref, vidx_ref):
    i = pl.program_id(0)

    @pl.when(i == 0)
    def _init():
        vmax_ref[:] = jnp.full((1, _ROWS), -jnp.inf, jnp.float32)
        vidx_ref[:] = jnp.zeros((1, _ROWS), jnp.int32)

    x = logits_ref[:]
    row = (i * _TILE_A
           + jax.lax.broadcasted_iota(jnp.int32, x.shape, 0))
    x = jnp.where(row < _VOCAB, x, -jnp.inf)
    tmax = jnp.max(x, axis=0, keepdims=True)
    tidx = jnp.min(jnp.where(x == tmax, row, _BIG_I32), axis=0, keepdims=True)
    upd = tmax > vmax_ref[:]
    vidx_ref[:] = jnp.where(upd, tidx, vidx_ref[:])
    vmax_ref[:] = jnp.maximum(tmax, vmax_ref[:])

    @pl.when(i == nva - 1)
    def _finish():
        target = vidx_ref[:]
        m = (dp_ref[:] == target).astype(jnp.float32)
        rp = jax.lax.broadcasted_iota(jnp.int32, (_ROWS, _ROWS), 0)
        r = jax.lax.broadcasted_iota(jnp.int32, (_ROWS, _ROWS), 1)
        tri = ((rp // _TPS == r // _TPS) & (rp <= r)).astype(jnp.float32)
        miss = jnp.dot(1.0 - m, tri, precision=_HIGH,
                       preferred_element_type=jnp.float32)
        prefix = (miss == 0.0).astype(jnp.float32)
        ra = jax.lax.broadcasted_iota(jnp.int32, (_ROWS, _BATCH), 0)
        ba = jax.lax.broadcasted_iota(jnp.int32, (_ROWS, _BATCH), 1)
        agg = ((ra // _TPS == ba) & (ra % _TPS < _L)).astype(jnp.float32)
        n_acc = 1 + jnp.dot(prefix, agg, precision=_HIGH,
                            preferred_element_type=jnp.float32).astype(jnp.int32)
        bidx = jax.lax.broadcasted_iota(jnp.int32, (1, _BATCH), 1)
        gid = _TPS * bidx + n_acc - 1
        oht = (ra == gid).astype(jnp.float32)
        last = jnp.dot(target.astype(jnp.float32), oht, precision=_HIGH,
                       preferred_element_type=jnp.float32)
        gid_ref[:] = gid
        nacc_ref[:] = n_acc
        last_ref[:] = last.astype(jnp.int32)


def _accept_recur_body(pmax_ref, pidx_ref, dp_ref, hs_ref, w_ref,
                       nacc_ref, last_ref, h_ref):
    pm = pmax_ref[:]
    gmax = jnp.max(pm, axis=0, keepdims=True)
    cand = jnp.where(pm == gmax, pidx_ref[:], _BIG_I32)
    target = jnp.min(cand, axis=0, keepdims=True)

    m = (dp_ref[:] == target).astype(jnp.float32)
    rp = jax.lax.broadcasted_iota(jnp.int32, (_ROWS, _ROWS), 0)
    r = jax.lax.broadcasted_iota(jnp.int32, (_ROWS, _ROWS), 1)
    tri = ((rp // _TPS == r // _TPS) & (rp <= r)).astype(jnp.float32)
    miss = jnp.dot(1.0 - m, tri, precision=_HIGH,
                   preferred_element_type=jnp.float32)
    prefix = (miss == 0.0).astype(jnp.float32)
    ra = jax.lax.broadcasted_iota(jnp.int32, (_ROWS, _BATCH), 0)
    ba = jax.lax.broadcasted_iota(jnp.int32, (_ROWS, _BATCH), 1)
    agg = ((ra // _TPS == ba) & (ra % _TPS < _L)).astype(jnp.float32)
    n_acc = 1 + jnp.dot(prefix, agg, precision=_HIGH,
                        preferred_element_type=jnp.float32).astype(jnp.int32)
    bidx = jax.lax.broadcasted_iota(jnp.int32, (1, _BATCH), 1)
    gid = _TPS * bidx + n_acc - 1
    oht = (ra == gid).astype(jnp.float32)
    last = jnp.dot(target.astype(jnp.float32), oht, precision=_HIGH,
                   preferred_element_type=jnp.float32)
    nacc_ref[:] = n_acc
    last_ref[:] = last.astype(jnp.int32)

    h = jax.lax.dot_general(oht, hs_ref[:], (((0,), (0,)), ((), ())),
                            precision=_HIGH,
                            preferred_element_type=jnp.float32)
    hs = []
    for _ in range(_L):
        h = jnp.tanh(jnp.dot(h, w_ref[:],
                             preferred_element_type=jnp.float32))
        hs.append(h)
    h_ref[:] = jnp.concatenate(hs, axis=0)


def _phase_b_body(nvb, h_in_ref, lm_ref, tok_ref, vmax_ref, vidx_ref):
    i = pl.program_id(0)

    @pl.when(i == 0)
    def _init():
        vmax_ref[:] = jnp.full((_L * _BATCH, 1), -jnp.inf, jnp.float32)
        vidx_ref[:] = jnp.zeros((_L * _BATCH, 1), jnp.int32)

    a = jax.lax.dot_general(h_in_ref[:], lm_ref[:], (((1,), (1,)), ((), ())),
                            preferred_element_type=jnp.float32)
    col = i * _TILE_B + jax.lax.broadcasted_iota(jnp.int32, a.shape, 1)
    a = jnp.where(col < _VOCAB, a, -jnp.inf)
    tmax = jnp.max(a, axis=1, keepdims=True)
    tidx = jnp.min(jnp.where(a == tmax, col, _BIG_I32), axis=1, keepdims=True)
    upd = tmax > vmax_ref[:]
    vidx_ref[:] = jnp.where(upd, tidx, vidx_ref[:])
    vmax_ref[:] = jnp.maximum(tmax, vmax_ref[:])

    @pl.when(i == nvb - 1)
    def _finish():
        tok_ref[:] = vidx_ref[:]


def kernel(logits, hidden_states, lm_head, W, draft_tokens):
    logits_t = logits.T
    lm_t = lm_head.T

    dp = jnp.concatenate(
        [draft_tokens, jnp.full((_BATCH, 1), -1, jnp.int32)], axis=1
    ).reshape(1, _ROWS)

    pmax, pidx = _sc_argmax(logits_t)

    n_acc, last, hmat = pl.pallas_call(
        _accept_recur_body,
        in_specs=[
            pl.BlockSpec((_NW, _ROWS), lambda: (0, 0)),
            pl.BlockSpec((_NW, _ROWS), lambda: (0, 0)),
            pl.BlockSpec((1, _ROWS), lambda: (0, 0)),
            pl.BlockSpec((_ROWS, _HID), lambda: (0, 0)),
            pl.BlockSpec((_HID, _HID), lambda: (0, 0)),
        ],
        out_specs=[
            pl.BlockSpec((1, _BATCH), lambda: (0, 0)),
            pl.BlockSpec((1, _BATCH), lambda: (0, 0)),
            pl.BlockSpec((_L * _BATCH, _HID), lambda: (0, 0)),
        ],
        out_shape=[
            jax.ShapeDtypeStruct((1, _BATCH), jnp.int32),
            jax.ShapeDtypeStruct((1, _BATCH), jnp.int32),
            jax.ShapeDtypeStruct((_L * _BATCH, _HID), jnp.float32),
        ],
    )(pmax, pidx, dp, hidden_states, W)

    nvb = pl.cdiv(_VOCAB, _TILE_B)
    tok = pl.pallas_call(
        functools.partial(_phase_b_body, nvb),
        grid=(nvb,),
        in_specs=[
            pl.BlockSpec((_L * _BATCH, _HID), lambda i: (0, 0)),
            pl.BlockSpec((_TILE_B, _HID), lambda i: (i, 0)),
        ],
        out_specs=pl.BlockSpec((_L * _BATCH, 1), lambda i: (0, 0)),
        out_shape=jax.ShapeDtypeStruct((_L * _BATCH, 1), jnp.int32),
        scratch_shapes=[
            pltpu.VMEM((_L * _BATCH, 1), jnp.float32),
            pltpu.VMEM((_L * _BATCH, 1), jnp.int32),
        ],
        compiler_params=pltpu.CompilerParams(
            dimension_semantics=("arbitrary",),
        ),
    )(hmat, lm_t)

    stacked = tok.reshape(_L, _BATCH).T
    next_new = jnp.concatenate([last.reshape(_BATCH, 1), stacked], axis=1)
    return next_new, stacked, n_acc.reshape(_BATCH)


# speedup vs baseline: 1.0666x; 1.0666x over previous
import functools

import jax
import jax.numpy as jnp
from jax import lax
from jax.experimental import pallas as pl
from jax.experimental.pallas import tpu as pltpu
from jax.experimental.pallas import tpu_sc as plsc

_BATCH = 32
_L = 3
_TPS = _L + 1
_ROWS = _BATCH * _TPS
_HID = 1024
_VOCAB = 100000

_TILE_A = 8192
_TILE_B = 4096

_HIGH = jax.lax.Precision.HIGHEST
_BIG_I32 = 2**30


_NW = 32
_TCV = 57344
_STRIPE = 1344
_CHUNK = 336
_NCH = _STRIPE // _CHUNK
_NK = 8
_VPB = _CHUNK // _NK


def _sc_argmax_body(lg_ref, pmax_ref, pidx_ref, buf0, buf1, rm_ref, ri_ref,
                    sem0, sem1):
    wid = lax.axis_index("s") * 2 + lax.axis_index("c")
    base = _TCV + wid * _STRIPE
    bufs = (buf0, buf1)
    sems = (sem0, sem1)

    def chunk_start(c):
        return jnp.minimum(base + c * _CHUNK, _VOCAB - _CHUNK)

    def issue(c):
        return pltpu.async_copy(
            lg_ref.at[pl.ds(chunk_start(c), _CHUNK)],
            bufs[c % 2], sems[c % 2])

    cp = issue(0)
    for c in range(_NCH):
        cp_next = issue(c + 1) if c + 1 < _NCH else None
        cp.wait()
        buf = bufs[c % 2]
        st_c = chunk_start(c)
        for cg in range(8):
            def body(v8, carry):
                ms, ixs = carry
                ms2, ixs2 = [], []
                for k in range(_NK):
                    row = v8 * _NK + k
                    x = buf[row, pl.ds(cg * 16, 16)]
                    vs = jnp.zeros((16,), jnp.int32) + (st_c + row)
                    upd = x > ms[k]
                    ms2.append(jnp.where(upd, x, ms[k]))
                    ixs2.append(jnp.where(upd, vs, ixs[k]))
                return (tuple(ms2), tuple(ixs2))

            init = (tuple(jnp.full((16,), -jnp.inf, jnp.float32)
                          for _ in range(_NK)),
                    tuple(jnp.zeros((16,), jnp.int32) for _ in range(_NK)))
            ms, ixs = jax.lax.fori_loop(0, _VPB, body, init)
            for k in range(_NK):
                off = ((c * 8 + cg) * _NK + k) * 16
                rm_ref[pl.ds(off, 16)] = ms[k]
                ri_ref[pl.ds(off, 16)] = ixs[k]
        cp = cp_next

    for cg in range(8):
        m = jnp.full((16,), -jnp.inf, jnp.float32)
        ix = jnp.zeros((16,), jnp.int32)
        for c in range(_NCH):
            for k in range(_NK):
                off = ((c * 8 + cg) * _NK + k) * 16
                m2 = rm_ref[pl.ds(off, 16)]
                i2 = ri_ref[pl.ds(off, 16)]
                gt = m2 > m
                eq = m2 == m
                ix = jnp.where(gt, i2, jnp.where(eq, jnp.minimum(ix, i2), ix))
                m = jnp.where(gt, m2, m)
        rm_ref[pl.ds(cg * 16, 16)] = m
        ri_ref[pl.ds(cg * 16, 16)] = ix

    pltpu.sync_copy(rm_ref.at[pl.ds(0, _ROWS)], pmax_ref.at[wid])
    pltpu.sync_copy(ri_ref.at[pl.ds(0, _ROWS)], pidx_ref.at[wid])


def _sc_argmax(logits_t):
    mesh = plsc.VectorSubcoreMesh(core_axis_name="c", subcore_axis_name="s")
    nacc = _NCH * 8 * _NK * 16
    f = functools.partial(
        pl.kernel,
        mesh=mesh,
        out_type=[
            jax.ShapeDtypeStruct((_NW, _ROWS), jnp.float32),
            jax.ShapeDtypeStruct((_NW, _ROWS), jnp.int32),
        ],
        scratch_types=[
            pltpu.VMEM((_CHUNK, _ROWS), jnp.float32),
            pltpu.VMEM((_CHUNK, _ROWS), jnp.float32),
            pltpu.VMEM((nacc,), jnp.float32),
            pltpu.VMEM((nacc,), jnp.int32),
            pltpu.SemaphoreType.DMA,
            pltpu.SemaphoreType.DMA,
        ],
    )(_sc_argmax_body)
    return f(logits_t)


def _tc_argmax_body(nva, logits_ref, tmax_ref, tidx_ref, vmax_ref, vidx_ref):
    i = pl.program_id(0)

    @pl.when(i == 0)
    def _init():
        vmax_ref[:] = jnp.full((1, _ROWS), -jnp.inf, jnp.float32)
        vidx_ref[:] = jnp.zeros((1, _ROWS), jnp.int32)

    x = logits_ref[:]
    row = (i * _TILE_A
           + jax.lax.broadcasted_iota(jnp.int32, x.shape, 0))
    tmax = jnp.max(x, axis=0, keepdims=True)
    tidx = jnp.min(jnp.where(x == tmax, row, _BIG_I32), axis=0, keepdims=True)
    upd = tmax > vmax_ref[:]
    vidx_ref[:] = jnp.where(upd, tidx, vidx_ref[:])
    vmax_ref[:] = jnp.maximum(tmax, vmax_ref[:])

    @pl.when(i == nva - 1)
    def _finish():
        tmax_ref[:] = vmax_ref[:]
        tidx_ref[:] = vidx_ref[:]

def _accept_recur_body(pmax_ref, pidx_ref, tmax_ref, tidx_ref, dp_ref,
                       hs_ref, w_ref, nacc_ref, last_ref, h_ref):
    pm = pmax_ref[:]
    gmax = jnp.maximum(jnp.max(pm, axis=0, keepdims=True), tmax_ref[:])
    cand = jnp.where(pm == gmax, pidx_ref[:], _BIG_I32)
    cand_tc = jnp.where(tmax_ref[:] == gmax, tidx_ref[:], _BIG_I32)
    target = jnp.minimum(jnp.min(cand, axis=0, keepdims=True), cand_tc)

    m = (dp_ref[:] == target).astype(jnp.float32)
    rp = jax.lax.broadcasted_iota(jnp.int32, (_ROWS, _ROWS), 0)
    r = jax.lax.broadcasted_iota(jnp.int32, (_ROWS, _ROWS), 1)
    tri = ((rp // _TPS == r // _TPS) & (rp <= r)).astype(jnp.float32)
    miss = jnp.dot(1.0 - m, tri, precision=_HIGH,
                   preferred_element_type=jnp.float32)
    prefix = (miss == 0.0).astype(jnp.float32)
    ra = jax.lax.broadcasted_iota(jnp.int32, (_ROWS, _BATCH), 0)
    ba = jax.lax.broadcasted_iota(jnp.int32, (_ROWS, _BATCH), 1)
    agg = ((ra // _TPS == ba) & (ra % _TPS < _L)).astype(jnp.float32)
    n_acc = 1 + jnp.dot(prefix, agg, precision=_HIGH,
                        preferred_element_type=jnp.float32).astype(jnp.int32)
    bidx = jax.lax.broadcasted_iota(jnp.int32, (1, _BATCH), 1)
    gid = _TPS * bidx + n_acc - 1
    oht = (ra == gid).astype(jnp.float32)
    last = jnp.dot(target.astype(jnp.float32), oht, precision=_HIGH,
                   preferred_element_type=jnp.float32)
    nacc_ref[:] = n_acc
    last_ref[:] = last.astype(jnp.int32)

    h = jax.lax.dot_general(oht, hs_ref[:], (((0,), (0,)), ((), ())),
                            precision=_HIGH,
                            preferred_element_type=jnp.float32)
    hs = []
    for _ in range(_L):
        h = jnp.tanh(jnp.dot(h, w_ref[:],
                             preferred_element_type=jnp.float32))
        hs.append(h)
    h_ref[:] = jnp.concatenate(hs, axis=0)


def _phase_b_body(nvb, h_in_ref, lm_ref, tok_ref, vmax_ref, vidx_ref):
    i = pl.program_id(0)

    @pl.when(i == 0)
    def _init():
        vmax_ref[:] = jnp.full((_L * _BATCH, 1), -jnp.inf, jnp.float32)
        vidx_ref[:] = jnp.zeros((_L * _BATCH, 1), jnp.int32)

    a = jax.lax.dot_general(h_in_ref[:], lm_ref[:], (((1,), (1,)), ((), ())),
                            preferred_element_type=jnp.float32)
    col = i * _TILE_B + jax.lax.broadcasted_iota(jnp.int32, a.shape, 1)
    a = jnp.where(col < _VOCAB, a, -jnp.inf)
    tmax = jnp.max(a, axis=1, keepdims=True)
    tidx = jnp.min(jnp.where(a == tmax, col, _BIG_I32), axis=1, keepdims=True)
    upd = tmax > vmax_ref[:]
    vidx_ref[:] = jnp.where(upd, tidx, vidx_ref[:])
    vmax_ref[:] = jnp.maximum(tmax, vmax_ref[:])

    @pl.when(i == nvb - 1)
    def _finish():
        tok_ref[:] = vidx_ref[:]


def kernel(logits, hidden_states, lm_head, W, draft_tokens):
    logits_t = logits.T
    lm_t = lm_head.T

    dp = jnp.concatenate(
        [draft_tokens, jnp.full((_BATCH, 1), -1, jnp.int32)], axis=1
    ).reshape(1, _ROWS)

    pmax, pidx = _sc_argmax(logits_t)

    nva = _TCV // _TILE_A
    tcmax, tcidx = pl.pallas_call(
        functools.partial(_tc_argmax_body, nva),
        grid=(nva,),
        in_specs=[pl.BlockSpec((_TILE_A, _ROWS), lambda i: (i, 0))],
        out_specs=[
            pl.BlockSpec((1, _ROWS), lambda i: (0, 0)),
            pl.BlockSpec((1, _ROWS), lambda i: (0, 0)),
        ],
        out_shape=[
            jax.ShapeDtypeStruct((1, _ROWS), jnp.float32),
            jax.ShapeDtypeStruct((1, _ROWS), jnp.int32),
        ],
        scratch_shapes=[
            pltpu.VMEM((1, _ROWS), jnp.float32),
            pltpu.VMEM((1, _ROWS), jnp.int32),
        ],
        compiler_params=pltpu.CompilerParams(
            dimension_semantics=("arbitrary",),
        ),
    )(logits_t)

    n_acc, last, hmat = pl.pallas_call(
        _accept_recur_body,
        in_specs=[
            pl.BlockSpec((_NW, _ROWS), lambda: (0, 0)),
            pl.BlockSpec((_NW, _ROWS), lambda: (0, 0)),
            pl.BlockSpec((1, _ROWS), lambda: (0, 0)),
            pl.BlockSpec((1, _ROWS), lambda: (0, 0)),
            pl.BlockSpec((1, _ROWS), lambda: (0, 0)),
            pl.BlockSpec((_ROWS, _HID), lambda: (0, 0)),
            pl.BlockSpec((_HID, _HID), lambda: (0, 0)),
        ],
        out_specs=[
            pl.BlockSpec((1, _BATCH), lambda: (0, 0)),
            pl.BlockSpec((1, _BATCH), lambda: (0, 0)),
            pl.BlockSpec((_L * _BATCH, _HID), lambda: (0, 0)),
        ],
        out_shape=[
            jax.ShapeDtypeStruct((1, _BATCH), jnp.int32),
            jax.ShapeDtypeStruct((1, _BATCH), jnp.int32),
            jax.ShapeDtypeStruct((_L * _BATCH, _HID), jnp.float32),
        ],
    )(pmax, pidx, tcmax, tcidx, dp, hidden_states, W)

    nvb = pl.cdiv(_VOCAB, _TILE_B)
    tok = pl.pallas_call(
        functools.partial(_phase_b_body, nvb),
        grid=(nvb,),
        in_specs=[
            pl.BlockSpec((_L * _BATCH, _HID), lambda i: (0, 0)),
            pl.BlockSpec((_TILE_B, _HID), lambda i: (i, 0)),
        ],
        out_specs=pl.BlockSpec((_L * _BATCH, 1), lambda i: (0, 0)),
        out_shape=jax.ShapeDtypeStruct((_L * _BATCH, 1), jnp.int32),
        scratch_shapes=[
            pltpu.VMEM((_L * _BATCH, 1), jnp.float32),
            pltpu.VMEM((_L * _BATCH, 1), jnp.int32),
        ],
        compiler_params=pltpu.CompilerParams(
            dimension_semantics=("arbitrary",),
        ),
    )(hmat, lm_t)

    stacked = tok.reshape(_L, _BATCH).T
    next_new = jnp.concatenate([last.reshape(_BATCH, 1), stacked], axis=1)
    return next_new, stacked, n_acc.reshape(_BATCH)
